# baseline (speedup 1.0000x reference)
import functools

import jax
import jax.numpy as jnp
from jax import lax
from jax.experimental import pallas as pl
from jax.experimental.pallas import tpu as pltpu
from jax.experimental.pallas import tpu_sc as plsc

_HIDDEN = 1024
_BATCH = 4
_SEQ = 2048
_TOK = _BATCH * _SEQ
_NW = 32
_PER_W = _TOK // _NW
_CH = 32
_NCH = _PER_W // _CH
_LANES = 16
_HG = _HIDDEN // _LANES

_mesh = plsc.VectorSubcoreMesh(core_axis_name="c", subcore_axis_name="s")


@functools.partial(
    pl.kernel,
    out_type=jax.ShapeDtypeStruct((_TOK, _HIDDEN), jnp.float32),
    mesh=_mesh,
    scratch_types=[
        pltpu.VMEM((_PER_W,), jnp.int32),
        pltpu.VMEM((_CH, _HIDDEN), jnp.float32),
        pltpu.VMEM((_CH, _HIDDEN), jnp.float32),
        pltpu.SemaphoreType.DMA,
        pltpu.SemaphoreType.DMA,
    ],
)
def _emb_kernel(idx_hbm, wtab_hbm, ptab_hbm, out_hbm, idx_v, wbuf, pbuf,
                gsem, psem):
    wid = lax.axis_index("s") * 2 + lax.axis_index("c")
    base = wid * _PER_W
    s0 = base % _SEQ
    pltpu.sync_copy(idx_hbm.at[pl.ds(base, _PER_W)], idx_v)

    def chunk(k, carry):
        r0 = k * _CH
        g = pltpu.async_copy(wtab_hbm.at[idx_v.at[pl.ds(r0, _CH)]], wbuf, gsem)
        p = pltpu.async_copy(ptab_hbm.at[pl.ds(s0 + r0, _CH)], pbuf, psem)
        g.wait()
        p.wait()

        def row(r, rcarry):
            for c in range(_HG):
                sl = pl.ds(c * _LANES, _LANES)
                plsc.addupdate(wbuf.at[r, sl], pbuf[r, sl])
            return rcarry

        lax.fori_loop(0, _CH, row, 0)
        pltpu.sync_copy(wbuf, out_hbm.at[pl.ds(base + r0, _CH)])
        return carry

    lax.fori_loop(0, _NCH, chunk, 0)


def kernel(inputs, word_embeddings, position_embeddings):
    flat_idx = inputs.reshape(_TOK).astype(jnp.int32)
    out = _emb_kernel(flat_idx, word_embeddings, position_embeddings)
    return out.reshape(_BATCH, _SEQ, _HIDDEN)


# speedup vs baseline: 1.1603x; 1.1603x over previous
import functools

import jax
import jax.numpy as jnp
from jax import lax
from jax.experimental import pallas as pl
from jax.experimental.pallas import tpu as pltpu
from jax.experimental.pallas import tpu_sc as plsc

_HIDDEN = 1024
_BATCH = 4
_SEQ = 2048
_TOK = _BATCH * _SEQ
_NW = 32
_SPW = _SEQ // _NW
_CH = 16
_CPB = _SPW // _CH
_NCH = _BATCH * _CPB
_LANES = 16
_HG = _HIDDEN // _LANES

_mesh = plsc.VectorSubcoreMesh(core_axis_name="c", subcore_axis_name="s")


@functools.partial(
    pl.kernel,
    out_type=jax.ShapeDtypeStruct((_TOK, _HIDDEN), jnp.float32),
    mesh=_mesh,
    scratch_types=[
        pltpu.VMEM((_BATCH * _SPW,), jnp.int32),
        pltpu.VMEM((_SPW, _HIDDEN), jnp.float32),
        pltpu.VMEM((_CH, _HIDDEN), jnp.float32),
        pltpu.VMEM((_CH, _HIDDEN), jnp.float32),
        pltpu.SemaphoreType.DMA,
        pltpu.SemaphoreType.DMA,
        pltpu.SemaphoreType.DMA,
        pltpu.SemaphoreType.DMA,
        pltpu.SemaphoreType.DMA,
    ],
)
def _emb_kernel(idx_hbm, wtab_hbm, ptab_hbm, out_hbm, idx_v, pos_v,
                wbuf0, wbuf1, gsem0, gsem1, osem0, osem1, psem):
    wid = lax.axis_index("s") * 2 + lax.axis_index("c")
    s0 = wid * _SPW
    wbufs = (wbuf0, wbuf1)
    gsems = (gsem0, gsem1)
    osems = (osem0, osem1)

    pos_desc = pltpu.async_copy(ptab_hbm.at[pl.ds(s0, _SPW)], pos_v, psem)
    for b in range(_BATCH):
        pltpu.sync_copy(idx_hbm.at[pl.ds(b * _SEQ + s0, _SPW)],
                        idx_v.at[pl.ds(b * _SPW, _SPW)])

    def gather(c, buf):
        return pltpu.async_copy(
            wtab_hbm.at[idx_v.at[pl.ds(c * _CH, _CH)]], wbufs[buf],
            gsems[buf])

    def writeback(c, buf):
        b, j = divmod(c, _CPB)
        row0 = b * _SEQ + s0 + j * _CH
        return pltpu.async_copy(wbufs[buf], out_hbm.at[pl.ds(row0, _CH)],
                                osems[buf])

    g_descs = [None] * _NCH
    o_descs = [None] * _NCH
    g_descs[0] = gather(0, 0)
    pos_desc.wait()
    for c in range(_NCH):
        buf = c % 2
        g_descs[c].wait()
        if c + 1 < _NCH:
            if c >= 1:
                o_descs[c - 1].wait()
            g_descs[c + 1] = gather(c + 1, 1 - buf)
        p0 = (c % _CPB) * _CH
        wb = wbufs[buf]

        def row(r, _, wb=wb, p0=p0):
            for g in range(_HG):
                sl = pl.ds(g * _LANES, _LANES)
                plsc.addupdate(wb.at[r, sl], pos_v[p0 + r, sl])
            return 0

        lax.fori_loop(0, _CH, row, 0)
        o_descs[c] = writeback(c, buf)
    o_descs[_NCH - 2].wait()
    o_descs[_NCH - 1].wait()


def kernel(inputs, word_embeddings, position_embeddings):
    flat_idx = inputs.reshape(_TOK).astype(jnp.int32)
    out = _emb_kernel(flat_idx, word_embeddings, position_embeddings)
    return out.reshape(_BATCH, _SEQ, _HIDDEN)
